# baseline (speedup 1.0000x reference)
import jax
import jax.numpy as jnp
from jax.experimental import pallas as pl
from jax.experimental.pallas import tpu as pltpu

B = 1024
H = 4096
R = 1024
CH = 512
BM = 128
NB = B // BM
KT = 4
RT = R // KT
HT = H // KT


def _dotT(a, b):
    return jax.lax.dot_general(
        a, b, (((1,), (1,)), ((), ())), preferred_element_type=jnp.float32)


def _fused(x_ref, w1_ref, b1_ref, w2_ref, b2_ref, dir_w_ref, dir_b_ref,
           ch_w1_ref, ch_b1_ref, w2d_ref, ssw_ref, vw_ref, sc_ref,
           tp1_ref, tpb1_ref, tp2_ref, tpb2_ref,
           npos_ref, scal_ref, latent_ref,
           xs, rs_s, dir_s, ch_s, tp1_s, tp2_s):
    bf16 = jnp.bfloat16
    kt = pl.program_id(0)
    b = pl.program_id(1)
    rows = pl.ds(b * BM, BM)

    @pl.when(b == 0)
    def _stash_tiles():
        tile = pl.ds(kt * RT, RT)
        dir_s[tile, :] = dir_w_ref[...].astype(bf16)
        ch_s[pl.ds(kt * (CH // KT), CH // KT), :] = ch_w1_ref[...].astype(bf16)
        tp1_s[tile, :] = tp1_ref[...].astype(bf16)
        tp2_s[pl.ds(kt * HT, HT), :] = tp2_ref[...].astype(bf16)

    @pl.when(kt == 0)
    def _stash_x():
        xs[rows, :] = x_ref[...].astype(bf16)

    x_bf = xs[rows, :]
    b1_t = b1_ref[:, pl.ds(kt * RT, RT)]
    h = jnp.maximum(_dotT(x_bf, w1_ref[...]) + b1_t, 0.0)
    partial = _dotT(h.astype(bf16), w2_ref[...])

    @pl.when(kt == 0)
    def _rs_init():
        rs_s[rows, :] = partial + b2_ref[...]

    @pl.when(kt > 0)
    def _rs_acc():
        rs_s[rows, :] += partial

    @pl.when(kt == KT - 1)
    def _epilogue():
        rs = rs_s[rows, :]
        rsb = rs.astype(bf16)

        dir_raw = _dotT(rsb, dir_s[...]) + dir_b_ref[...]
        ch_h = jnp.maximum(_dotT(rsb, ch_s[...]) + ch_b1_ref[...], 0.0)

        sc = sc_ref[...]
        d = jnp.sum(ch_h * w2d_ref[...], axis=1, keepdims=True) + sc[0, 2]
        ss_logit = jnp.sum(rs * ssw_ref[...], axis=1, keepdims=True) + sc[0, 0]
        value = jnp.sum(rs * vw_ref[...], axis=1, keepdims=True) + sc[0, 1]

        p0 = jax.nn.sigmoid(d)
        p1 = jax.nn.sigmoid(-d)
        entropy = -(p0 * jnp.log(p0 + 1e-8) + p1 * jnp.log(p1 + 1e-8))
        log_prob = jax.nn.log_sigmoid(jnp.abs(d))

        norm = jnp.maximum(
            jnp.sqrt(jnp.sum(dir_raw * dir_raw, axis=1, keepdims=True)),
            1e-12)
        step = jax.nn.sigmoid(ss_logit) * 2.0
        npos = rs + (step / norm) * dir_raw

        h2 = jnp.maximum(
            _dotT(npos.astype(bf16), tp1_s[...]) + tpb1_ref[...], 0.0)
        latent_ref[...] = _dotT(h2.astype(bf16), tp2_s[...]) + tpb2_ref[...]
        npos_ref[...] = npos
        scal_ref[...] = jnp.concatenate([p0, value, log_prob, entropy],
                                        axis=1)


def kernel(state, step_num, sp_w1, sp_b1, sp_w2, sp_b2, tp_w1, tp_b1,
           tp_w2, tp_b2, ch_w1, ch_b1, ch_w2, ch_b2, dir_w, dir_b,
           ss_w, ss_b, v_w, v_b):
    f32 = jnp.float32
    bf16 = jnp.bfloat16
    shift = 0.1 * jnp.sin(jnp.float32(step_num) * 0.5)

    b2 = (sp_b2 + shift)[None, :]
    w2d = (ch_w2[0] - ch_w2[1])[None, :]
    scalars = jnp.stack(
        [ss_b[0], v_b[0], ch_b2[0] - ch_b2[1]])[None, :]

    grid = (KT, NB)
    const = lambda shape: pl.BlockSpec(shape, lambda kt, b: (0, 0))
    phase = lambda shape: pl.BlockSpec(shape, lambda kt, b: (kt, 0))
    phase_c = lambda shape: pl.BlockSpec(shape, lambda kt, b: (0, kt))
    out_sp = lambda shape: pl.BlockSpec(
        shape, lambda kt, b: (jnp.where(kt == KT - 1, b, 0), 0))

    npos, scal, latent = pl.pallas_call(
        _fused,
        grid=grid,
        in_specs=[
            pl.BlockSpec((BM, H),
                         lambda kt, b: (jnp.where(kt == 0, b, NB - 1), 0)),
            phase((RT, H)), const((1, R)),
            phase_c((R, RT)), const((1, R)),
            phase((RT, R)), const((1, R)),
            phase((CH // KT, R)), const((1, CH)),
            const((1, CH)),
            const((1, R)), const((1, R)),
            const((1, 3)),
            phase((RT, R)), const((1, R)),
            phase((HT, R)), const((1, H)),
        ],
        out_specs=[
            out_sp((BM, R)),
            out_sp((BM, 4)),
            out_sp((BM, H)),
        ],
        out_shape=[
            jax.ShapeDtypeStruct((B, R), f32),
            jax.ShapeDtypeStruct((B, 4), f32),
            jax.ShapeDtypeStruct((B, H), f32),
        ],
        scratch_shapes=[
            pltpu.VMEM((B, H), bf16),
            pltpu.VMEM((B, R), f32),
            pltpu.VMEM((R, R), bf16),
            pltpu.VMEM((CH, R), bf16),
            pltpu.VMEM((R, R), bf16),
            pltpu.VMEM((H, R), bf16),
        ],
        compiler_params=pltpu.CompilerParams(
            dimension_semantics=("arbitrary", "arbitrary"),
            vmem_limit_bytes=64 * 1024 * 1024,
        ),
    )(state, sp_w1, sp_b1[None, :], sp_w2, b2, dir_w, dir_b[None, :],
      ch_w1, ch_b1[None, :], w2d, ss_w, v_w, scalars,
      tp_w1, tp_b1[None, :], tp_w2, tp_b2[None, :])

    return (latent, npos, scal[:, 0], scal[:, 1], scal[:, 2], scal[:, 3])


# speedup vs baseline: 1.8604x; 1.8604x over previous
import jax
import jax.numpy as jnp
from jax.experimental import pallas as pl
from jax.experimental.pallas import tpu as pltpu

B = 1024
H = 4096
R = 1024
CH = 512
XC = 128
WC = 512


def _dotT(a, b):
    return jax.lax.dot_general(
        a, b, (((1,), (1,)), ((), ())), preferred_element_type=jnp.float32)


def _fused(x_hbm, w1_hbm, w2_hbm, dir_hbm, ch_hbm, tp1_hbm, tp2_hbm,
           b1_ref, b2_ref, dir_b_ref, ch_b1_ref, w2d_ref, ssw_ref, vw_ref,
           sc_ref, tpb1_ref, tpb2_ref,
           npos_hbm, scal_hbm, latent_hbm,
           sa, sb, xs, w1s, w2s, dirs, chs, tp1s, tp2s,
           h_s, rs_s, rsb_s, dir_s, lat_s, scal_s,
           sa_sem, sb_sem, out_sem, lat_sem):
    bf16 = jnp.bfloat16

    sa_stream = ([(x_hbm, c) for c in range(B // XC)]
                 + [(w1_hbm, c) for c in range(R // XC)])

    def sa_copy(i, buf):
        ref, c = sa_stream[i]
        return pltpu.make_async_copy(
            ref.at[pl.ds(c * XC, XC), :], sa.at[buf], sa_sem.at[buf])

    sb_stream = ([(w2_hbm, w2s, c) for c in range(R // WC)]
                 + [(dir_hbm, dirs, c) for c in range(R // WC)]
                 + [(ch_hbm, chs, 0)]
                 + [(tp1_hbm, tp1s, c) for c in range(R // WC)]
                 + [(tp2_hbm, tp2s, c) for c in range(H // WC)])

    def sb_copy(i, buf):
        ref, _, c = sb_stream[i]
        return pltpu.make_async_copy(
            ref.at[pl.ds(c * WC, WC), :], sb.at[buf], sb_sem.at[buf])

    def sb_start(i):
        if i < len(sb_stream):
            sb_copy(i, i % 2).start()

    def sb_landed(i):
        _, dst, c = sb_stream[i]
        sb_copy(i, i % 2).wait()
        dst[pl.ds(c * WC, WC), :] = sb[i % 2].astype(bf16)
        sb_start(i + 2)

    sa_copy(0, 0).start()
    sa_copy(1, 1).start()
    n_sa = len(sa_stream)
    for i in range(n_sa):
        ref, c = sa_stream[i]
        sa_copy(i, i % 2).wait()
        dst = xs if ref is x_hbm else w1s
        dst[pl.ds(c * XC, XC), :] = sa[i % 2].astype(bf16)
        if i + 2 < n_sa:
            sa_copy(i + 2, i % 2).start()
    sb_start(0)
    sb_start(1)

    h_s[...] = jnp.maximum(
        _dotT(xs[...], w1s[...]) + b1_ref[...], 0.0).astype(bf16)

    si = 0
    for _ in range(R // WC):
        sb_landed(si)
        si += 1
    rs = _dotT(h_s[...], w2s[...]) + b2_ref[...]
    rs_s[...] = rs
    rsb_s[...] = rs.astype(bf16)

    for _ in range(R // WC):
        sb_landed(si)
        si += 1
    dir_s[...] = _dotT(rsb_s[...], dirs[...]) + dir_b_ref[...]

    sb_landed(si)
    si += 1
    ch_h = jnp.maximum(_dotT(rsb_s[...], chs[...]) + ch_b1_ref[...], 0.0)

    rs = rs_s[...]
    sc = sc_ref[...]
    d = jnp.sum(ch_h * w2d_ref[...], axis=1, keepdims=True) + sc[0, 2]
    ss_logit = jnp.sum(rs * ssw_ref[...], axis=1, keepdims=True) + sc[0, 0]
    value = jnp.sum(rs * vw_ref[...], axis=1, keepdims=True) + sc[0, 1]

    p0 = jax.nn.sigmoid(d)
    p1 = jax.nn.sigmoid(-d)
    entropy = -(p0 * jnp.log(p0 + 1e-8) + p1 * jnp.log(p1 + 1e-8))
    log_prob = jax.nn.log_sigmoid(jnp.abs(d))

    dir_raw = dir_s[...]
    norm = jnp.maximum(
        jnp.sqrt(jnp.sum(dir_raw * dir_raw, axis=1, keepdims=True)), 1e-12)
    step = jax.nn.sigmoid(ss_logit) * 2.0
    npos = rs + (step / norm) * dir_raw

    rs_s[...] = npos
    rsb_s[...] = npos.astype(bf16)
    pltpu.make_async_copy(rs_s, npos_hbm, out_sem.at[0]).start()
    scal_s[...] = jnp.concatenate([p0, value, log_prob, entropy], axis=1)
    pltpu.make_async_copy(scal_s, scal_hbm, out_sem.at[1]).start()

    for _ in range(R // WC):
        sb_landed(si)
        si += 1
    h_s[...] = jnp.maximum(
        _dotT(rsb_s[...], tp1s[...]) + tpb1_ref[...], 0.0).astype(bf16)

    nlat = H // WC
    for c in range(nlat):
        sb_landed(si)
        si += 1
        if c >= 2:
            pltpu.make_async_copy(
                lat_s.at[c % 2], latent_hbm.at[:, pl.ds((c - 2) * WC, WC)],
                lat_sem.at[c % 2]).wait()
        cols = pl.ds(c * WC, WC)
        lat_s[c % 2] = _dotT(
            h_s[...], tp2s[cols, :]) + tpb2_ref[:, cols]
        pltpu.make_async_copy(
            lat_s.at[c % 2], latent_hbm.at[:, cols], lat_sem.at[c % 2]).start()

    for c in (nlat - 2, nlat - 1):
        pltpu.make_async_copy(
            lat_s.at[c % 2], latent_hbm.at[:, pl.ds(c * WC, WC)],
            lat_sem.at[c % 2]).wait()
    pltpu.make_async_copy(rs_s, npos_hbm, out_sem.at[0]).wait()
    pltpu.make_async_copy(scal_s, scal_hbm, out_sem.at[1]).wait()


def kernel(state, step_num, sp_w1, sp_b1, sp_w2, sp_b2, tp_w1, tp_b1,
           tp_w2, tp_b2, ch_w1, ch_b1, ch_w2, ch_b2, dir_w, dir_b,
           ss_w, ss_b, v_w, v_b):
    f32 = jnp.float32
    bf16 = jnp.bfloat16
    shift = 0.1 * jnp.sin(jnp.float32(step_num) * 0.5)

    b2 = (sp_b2 + shift)[None, :]
    w2d = (ch_w2[0] - ch_w2[1])[None, :]
    scalars = jnp.stack(
        [ss_b[0], v_b[0], ch_b2[0] - ch_b2[1]])[None, :]

    anyspec = pl.BlockSpec(memory_space=pl.ANY)
    vmem = pl.BlockSpec(memory_space=pltpu.MemorySpace.VMEM)

    npos, scal, latent = pl.pallas_call(
        _fused,
        in_specs=[anyspec] * 7 + [vmem] * 10,
        out_specs=[anyspec, anyspec, anyspec],
        out_shape=[
            jax.ShapeDtypeStruct((B, R), f32),
            jax.ShapeDtypeStruct((B, 4), f32),
            jax.ShapeDtypeStruct((B, H), f32),
        ],
        scratch_shapes=[
            pltpu.VMEM((2, XC, H), f32),
            pltpu.VMEM((2, WC, R), f32),
            pltpu.VMEM((B, H), bf16),
            pltpu.VMEM((R, H), bf16),
            pltpu.VMEM((R, R), bf16),
            pltpu.VMEM((R, R), bf16),
            pltpu.VMEM((CH, R), bf16),
            pltpu.VMEM((R, R), bf16),
            pltpu.VMEM((H, R), bf16),
            pltpu.VMEM((B, R), bf16),
            pltpu.VMEM((B, R), f32),
            pltpu.VMEM((B, R), bf16),
            pltpu.VMEM((B, R), f32),
            pltpu.VMEM((2, B, WC), f32),
            pltpu.VMEM((B, 4), f32),
            pltpu.SemaphoreType.DMA((2,)),
            pltpu.SemaphoreType.DMA((2,)),
            pltpu.SemaphoreType.DMA((2,)),
            pltpu.SemaphoreType.DMA((2,)),
        ],
        compiler_params=pltpu.CompilerParams(
            vmem_limit_bytes=64 * 1024 * 1024,
        ),
    )(state, sp_w1, sp_w2, dir_w, ch_w1, tp_w1, tp_w2,
      sp_b1[None, :], b2, dir_b[None, :], ch_b1[None, :], w2d, ss_w, v_w,
      scalars, tp_b1[None, :], tp_b2[None, :])

    return (latent, npos, scal[:, 0], scal[:, 1], scal[:, 2], scal[:, 3])


# speedup vs baseline: 2.0410x; 1.0971x over previous
import jax
import jax.numpy as jnp
from jax.experimental import pallas as pl
from jax.experimental.pallas import tpu as pltpu

B = 1024
H = 4096
R = 1024
CH = 512
XC = 128
WC = 512
NSA = 4
NSB = 4


def _dotT(a, b):
    return jax.lax.dot_general(
        a, b, (((1,), (1,)), ((), ())), preferred_element_type=jnp.float32)


def _fused(x_hbm, w1_hbm, w2_hbm, dir_hbm, ch_hbm, tp1_hbm, tp2_hbm,
           b1_ref, b2_ref, dir_b_ref, ch_b1_ref, w2d_ref, ssw_ref, vw_ref,
           sc_ref, tpb1_ref, tpb2_ref,
           npos_hbm, scal_hbm, latent_hbm,
           sa, sb, xs, w1s, h_s, rs_s, rsb_s, dir_s, lat_s, scal_s,
           sa_sem, sb_sem, out_sem, lat_sem):
    bf16 = jnp.bfloat16

    sa_stream = ([(x_hbm, c) for c in range(B // XC)]
                 + [(w1_hbm, c) for c in range(R // XC)])

    def sa_copy(i):
        ref, c = sa_stream[i]
        return pltpu.make_async_copy(
            ref.at[pl.ds(c * XC, XC), :], sa.at[i % NSA], sa_sem.at[i % NSA])

    sb_stream = ([(w2_hbm, c) for c in range(R // WC)]
                 + [(dir_hbm, c) for c in range(R // WC)]
                 + [(ch_hbm, 0)]
                 + [(tp1_hbm, c) for c in range(R // WC)]
                 + [(tp2_hbm, c) for c in range(H // WC)])

    def sb_copy(i):
        ref, c = sb_stream[i]
        return pltpu.make_async_copy(
            ref.at[pl.ds(c * WC, WC), :], sb.at[i % NSB], sb_sem.at[i % NSB])

    def sb_start(i):
        if i < len(sb_stream):
            sb_copy(i).start()

    def sb_take(i):
        sb_copy(i).wait()
        w = sb[i % NSB].astype(bf16)
        sb_start(i + NSB)
        return w

    n_sa = len(sa_stream)
    for i in range(min(NSA, n_sa)):
        sa_copy(i).start()
    for i in range(n_sa):
        ref, c = sa_stream[i]
        sa_copy(i).wait()
        dst = xs if ref is x_hbm else w1s
        dst[pl.ds(c * XC, XC), :] = sa[i % NSA].astype(bf16)
        if i + NSA < n_sa:
            sa_copy(i + NSA).start()
    for i in range(NSB):
        sb_start(i)

    h_s[...] = jnp.maximum(
        _dotT(xs[...], w1s[...]) + b1_ref[...], 0.0).astype(bf16)

    si = 0
    for c in range(R // WC):
        cols = pl.ds(c * WC, WC)
        rs_s[:, cols] = _dotT(h_s[...], sb_take(si)) + b2_ref[:, cols]
        si += 1
    rsb_s[...] = rs_s[...].astype(bf16)

    for c in range(R // WC):
        cols = pl.ds(c * WC, WC)
        dir_s[:, cols] = _dotT(rsb_s[...], sb_take(si)) + dir_b_ref[:, cols]
        si += 1

    ch_h = jnp.maximum(_dotT(rsb_s[...], sb_take(si)) + ch_b1_ref[...], 0.0)
    si += 1

    rs = rs_s[...]
    sc = sc_ref[...]
    d = jnp.sum(ch_h * w2d_ref[...], axis=1, keepdims=True) + sc[0, 2]
    ss_logit = jnp.sum(rs * ssw_ref[...], axis=1, keepdims=True) + sc[0, 0]
    value = jnp.sum(rs * vw_ref[...], axis=1, keepdims=True) + sc[0, 1]

    p0 = jax.nn.sigmoid(d)
    p1 = jax.nn.sigmoid(-d)
    entropy = -(p0 * jnp.log(p0 + 1e-8) + p1 * jnp.log(p1 + 1e-8))
    log_prob = jax.nn.log_sigmoid(jnp.abs(d))

    dir_raw = dir_s[...]
    norm = jnp.maximum(
        jnp.sqrt(jnp.sum(dir_raw * dir_raw, axis=1, keepdims=True)), 1e-12)
    step = jax.nn.sigmoid(ss_logit) * 2.0
    npos = rs + (step / norm) * dir_raw

    rs_s[...] = npos
    rsb_s[...] = npos.astype(bf16)
    pltpu.make_async_copy(rs_s, npos_hbm, out_sem.at[0]).start()
    scal_s[...] = jnp.concatenate([p0, value, log_prob, entropy], axis=1)
    pltpu.make_async_copy(scal_s, scal_hbm, out_sem.at[1]).start()

    for c in range(R // WC):
        cols = pl.ds(c * WC, WC)
        h_s[:, cols] = jnp.maximum(
            _dotT(rsb_s[...], sb_take(si)) + tpb1_ref[:, cols],
            0.0).astype(bf16)
        si += 1

    nlat = H // WC
    for c in range(nlat):
        w = sb_take(si)
        si += 1
        if c >= 2:
            pltpu.make_async_copy(
                lat_s.at[c % 2], latent_hbm.at[:, pl.ds((c - 2) * WC, WC)],
                lat_sem.at[c % 2]).wait()
        cols = pl.ds(c * WC, WC)
        lat_s[c % 2] = _dotT(h_s[...], w) + tpb2_ref[:, cols]
        pltpu.make_async_copy(
            lat_s.at[c % 2], latent_hbm.at[:, cols], lat_sem.at[c % 2]).start()

    for c in (nlat - 2, nlat - 1):
        pltpu.make_async_copy(
            lat_s.at[c % 2], latent_hbm.at[:, pl.ds(c * WC, WC)],
            lat_sem.at[c % 2]).wait()
    pltpu.make_async_copy(rs_s, npos_hbm, out_sem.at[0]).wait()
    pltpu.make_async_copy(scal_s, scal_hbm, out_sem.at[1]).wait()


def kernel(state, step_num, sp_w1, sp_b1, sp_w2, sp_b2, tp_w1, tp_b1,
           tp_w2, tp_b2, ch_w1, ch_b1, ch_w2, ch_b2, dir_w, dir_b,
           ss_w, ss_b, v_w, v_b):
    f32 = jnp.float32
    bf16 = jnp.bfloat16
    shift = 0.1 * jnp.sin(jnp.float32(step_num) * 0.5)

    b2 = (sp_b2 + shift)[None, :]
    w2d = (ch_w2[0] - ch_w2[1])[None, :]
    scalars = jnp.stack(
        [ss_b[0], v_b[0], ch_b2[0] - ch_b2[1]])[None, :]

    anyspec = pl.BlockSpec(memory_space=pl.ANY)
    vmem = pl.BlockSpec(memory_space=pltpu.MemorySpace.VMEM)

    npos, scal, latent = pl.pallas_call(
        _fused,
        in_specs=[anyspec] * 7 + [vmem] * 10,
        out_specs=[anyspec, anyspec, anyspec],
        out_shape=[
            jax.ShapeDtypeStruct((B, R), f32),
            jax.ShapeDtypeStruct((B, 4), f32),
            jax.ShapeDtypeStruct((B, H), f32),
        ],
        scratch_shapes=[
            pltpu.VMEM((NSA, XC, H), f32),
            pltpu.VMEM((NSB, WC, R), f32),
            pltpu.VMEM((B, H), bf16),
            pltpu.VMEM((R, H), bf16),
            pltpu.VMEM((B, R), bf16),
            pltpu.VMEM((B, R), f32),
            pltpu.VMEM((B, R), bf16),
            pltpu.VMEM((B, R), f32),
            pltpu.VMEM((2, B, WC), f32),
            pltpu.VMEM((B, 4), f32),
            pltpu.SemaphoreType.DMA((NSA,)),
            pltpu.SemaphoreType.DMA((NSB,)),
            pltpu.SemaphoreType.DMA((2,)),
            pltpu.SemaphoreType.DMA((2,)),
        ],
        compiler_params=pltpu.CompilerParams(
            vmem_limit_bytes=64 * 1024 * 1024,
        ),
    )(state, sp_w1, sp_w2, dir_w, ch_w1, tp_w1, tp_w2,
      sp_b1[None, :], b2, dir_b[None, :], ch_b1[None, :], w2d, ss_w, v_w,
      scalars, tp_b1[None, :], tp_b2[None, :])

    return (latent, npos, scal[:, 0], scal[:, 1], scal[:, 2], scal[:, 3])
